# baseline (speedup 1.0000x reference)
import jax
import jax.numpy as jnp
from jax.experimental import pallas as pl
from jax.experimental.pallas import tpu as pltpu

_BLOCK = 128
_GROUPS = 2
_GBLK = _BLOCK * _GROUPS
_IN_ROWS = 1184
_C1_ROWS = 1024
_P1_ROWS = 256
_C2_ROWS = 160
_P2_ROWS = 40
_M1 = 6 * 896
_K1 = 1056


def _kern(x_ref, w1_ref, b1_ref, w2d_ref, b2_ref,
          wfc1_ref, bfc1_ref, wfc2_ref, bfc2_ref, wfc3_ref, bfc3_ref,
          out_ref, a1_ref, p1_ref, a2_ref, p2_ref, xsh_ref):
    f32 = jnp.float32
    for lg in range(_GROUPS):
        _lane_group(x_ref, w1_ref, b1_ref, w2d_ref, b2_ref,
                    wfc1_ref, bfc1_ref, wfc2_ref, bfc2_ref, wfc3_ref,
                    bfc3_ref, out_ref, a1_ref, p1_ref, a2_ref, p2_ref,
                    xsh_ref, lg)


def _lane_group(x_ref, w1_ref, b1_ref, w2d_ref, b2_ref,
                wfc1_ref, bfc1_ref, wfc2_ref, bfc2_ref, wfc3_ref, bfc3_ref,
                out_ref, a1_ref, p1_ref, a2_ref, p2_ref, xsh_ref, lg):
    f32 = jnp.float32
    lo = lg * _BLOCK

    for p in range(1, 5):
        xsh_ref[(p - 1) * _IN_ROWS:(p - 1) * _IN_ROWS + 1032, :] = \
            x_ref[p:p + 1032, lo:lo + _BLOCK]

    for cg in range(3):
        c0, c1 = 2 * cg, 2 * cg + 1
        for ch in range(7):
            base = ch * 128
            acc0 = jnp.full((128, _BLOCK), b1_ref[0, c0], f32)
            acc1 = jnp.full((128, _BLOCK), b1_ref[0, c1], f32)
            for di in range(5):
                for dj in range(5):
                    k = di * 5 + dj
                    if dj == 0:
                        xs = x_ref[base + di * 32:base + di * 32 + 128,
                                   lo:lo + _BLOCK]
                    else:
                        o = (dj - 1) * _IN_ROWS + base + di * 32
                        xs = xsh_ref[o:o + 128, :]
                    acc0 = acc0 + xs * w1_ref[k, c0]
                    acc1 = acc1 + xs * w1_ref[k, c1]
            a1_ref[base:base + 128, :] = acc0
            a1_ref[_C1_ROWS + base:_C1_ROWS + base + 128, :] = acc1
        for ci in range(2):
            ab = ci * _C1_ROWS
            pb = (2 * cg + ci) * _P1_ROWS
            p1_ref[pb + 224:pb + 256, :] = jnp.zeros((32, _BLOCK),
                                                     jnp.bfloat16)
            for m in range(14):
                r00 = a1_ref[pl.ds(ab + 2 * m * 32, 16, 2), :]
                r01 = a1_ref[pl.ds(ab + 2 * m * 32 + 1, 16, 2), :]
                r10 = a1_ref[pl.ds(ab + (2 * m + 1) * 32, 16, 2), :]
                r11 = a1_ref[pl.ds(ab + (2 * m + 1) * 32 + 1, 16, 2), :]
                p1_ref[pb + m * 16:pb + (m + 1) * 16, :] = jnp.maximum(
                    jnp.maximum(jnp.maximum(r00, r01), jnp.maximum(r10, r11)),
                    0.0).astype(jnp.bfloat16)

    a2_ref[...] = jax.lax.dot_general(
        w2d_ref[...], p1_ref[...], (((1,), (0,)), ((), ())),
        preferred_element_type=f32)

    for co in range(16):
        qb = co * _C2_ROWS
        for m in range(5):
            r00 = a2_ref[pl.ds(qb + 2 * m * 16, 8, 2), :]
            r01 = a2_ref[pl.ds(qb + 2 * m * 16 + 1, 8, 2), :]
            r10 = a2_ref[pl.ds(qb + (2 * m + 1) * 16, 8, 2), :]
            r11 = a2_ref[pl.ds(qb + (2 * m + 1) * 16 + 1, 8, 2), :]
            p2_ref[co * _P2_ROWS + m * 8:co * _P2_ROWS + (m + 1) * 8, :] = \
                jnp.maximum(
                    jnp.maximum(jnp.maximum(r00, r01),
                                jnp.maximum(r10, r11)) + b2_ref[0, co],
                    0.0)

    p2 = p2_ref[...]
    h = jax.lax.dot_general(p2, wfc1_ref[...], (((0,), (0,)), ((), ())),
                            preferred_element_type=f32)
    h = jnp.maximum(h + bfc1_ref[...], 0.0)
    h = jnp.maximum(
        jnp.dot(h, wfc2_ref[...], preferred_element_type=f32) + bfc2_ref[...],
        0.0)
    out_ref[lo:lo + _BLOCK, :] = (
        jnp.dot(h, wfc3_ref[...], preferred_element_type=f32) + bfc3_ref[...])


def kernel(w1, b1, w2, b2, wfc1, bfc1, wfc2, bfc2, wfc3, bfc3, x_nchw):
    n = x_nchw.shape[0]
    nb = (n + _GBLK - 1) // _GBLK
    npad = nb * _GBLK

    x = x_nchw.reshape(n, 28, 28).astype(jnp.float32)
    xp = jnp.pad(x, ((0, npad - n), (2, 2), (2, 2)))
    xT = jnp.pad(xp.reshape(npad, 1024).T, ((0, 160), (0, 0)))

    wf = jnp.transpose(wfc1.reshape(5, 5, 16, 128), (2, 0, 1, 3))
    wf = jnp.pad(wf, ((0, 0), (0, 0), (0, 3), (0, 0))).reshape(640, 128)

    eyes = jnp.stack([jnp.eye(_C2_ROWS, _P1_ROWS, k=di * 16 + dj,
                              dtype=jnp.float32)
                      for di in range(5) for dj in range(5)])
    w2d = jnp.einsum("tic,trs->cris", w2[:, :6, :], eyes)
    w2d = w2d.reshape(16 * _C2_ROWS, 6 * _P1_ROWS)
    w2d = w2d.astype(jnp.bfloat16)

    smem = pl.BlockSpec(memory_space=pltpu.SMEM)

    def _wspec(shp):
        return pl.BlockSpec(shp, lambda i, _s=shp: (0,) * len(_s))

    out = pl.pallas_call(
        _kern,
        grid=(nb,),
        out_shape=jax.ShapeDtypeStruct((npad, 128), jnp.float32),
        in_specs=[
            pl.BlockSpec((_IN_ROWS, _GBLK), lambda i: (0, i)),
            smem,
            smem,
            _wspec((2560, 1536)),
            smem,
            _wspec((640, 128)),
            _wspec((1, 128)),
            _wspec((128, 128)),
            _wspec((1, 128)),
            _wspec((128, 128)),
            _wspec((1, 128)),
        ],
        out_specs=pl.BlockSpec((_GBLK, 128), lambda i: (i, 0)),
        scratch_shapes=[
            pltpu.VMEM((2 * _C1_ROWS, _BLOCK), jnp.float32),
            pltpu.VMEM((6 * _P1_ROWS, _BLOCK), jnp.bfloat16),
            pltpu.VMEM((16 * _C2_ROWS, _BLOCK), jnp.float32),
            pltpu.VMEM((16 * _P2_ROWS, _BLOCK), jnp.float32),
            pltpu.VMEM((4 * _IN_ROWS, _BLOCK), jnp.float32),
        ],
        compiler_params=pltpu.CompilerParams(
            dimension_semantics=("parallel",)),
    )(xT, w1, b1, w2d, b2, wf, bfc1, wfc2, bfc2, wfc3, bfc3)
    return out[:n, :10]


# speedup vs baseline: 1.1195x; 1.1195x over previous
import jax
import jax.numpy as jnp
from jax.experimental import pallas as pl
from jax.experimental.pallas import tpu as pltpu

_BLOCK = 128
_IN_ROWS = 1184
_C1_ROWS = 1024
_P1_ROWS = 256
_C2_ROWS = 160
_P2_ROWS = 40
_M1 = 6 * 896
_K1 = 1056


def _kern(x_ref, w1_ref, b1_ref, w2d_ref, b2_ref,
          wfc1_ref, bfc1_ref, wfc2_ref, bfc2_ref, wfc3_ref, bfc3_ref,
          out_ref, a1_ref, p1_ref, a2_ref, p2_ref):
    f32 = jnp.float32

    for cg in range(3):
        c0, c1 = 2 * cg, 2 * cg + 1
        for ch in range(7):
            base = ch * 128
            acc0 = jnp.full((128, _BLOCK), b1_ref[0, c0], f32)
            acc1 = jnp.full((128, _BLOCK), b1_ref[0, c1], f32)
            for di in range(5):
                for dj in range(5):
                    k = di * 5 + dj
                    off = base + di * 32 + dj
                    xs = x_ref[off:off + 128, :]
                    acc0 = acc0 + xs * w1_ref[k, c0]
                    acc1 = acc1 + xs * w1_ref[k, c1]
            a1_ref[base:base + 128, :] = acc0
            a1_ref[_C1_ROWS + base:_C1_ROWS + base + 128, :] = acc1
        for ci in range(2):
            ab = ci * _C1_ROWS
            pb = (2 * cg + ci) * _P1_ROWS
            p1_ref[pb + 224:pb + 256, :] = jnp.zeros((32, _BLOCK),
                                                     jnp.bfloat16)
            for m in range(14):
                r00 = a1_ref[pl.ds(ab + 2 * m * 32, 16, 2), :]
                r01 = a1_ref[pl.ds(ab + 2 * m * 32 + 1, 16, 2), :]
                r10 = a1_ref[pl.ds(ab + (2 * m + 1) * 32, 16, 2), :]
                r11 = a1_ref[pl.ds(ab + (2 * m + 1) * 32 + 1, 16, 2), :]
                p1_ref[pb + m * 16:pb + (m + 1) * 16, :] = jnp.maximum(
                    jnp.maximum(jnp.maximum(r00, r01), jnp.maximum(r10, r11)),
                    0.0).astype(jnp.bfloat16)

    a2_ref[...] = jax.lax.dot_general(
        w2d_ref[...], p1_ref[...], (((1,), (0,)), ((), ())),
        preferred_element_type=f32)

    for co in range(16):
        qb = co * _C2_ROWS
        for m in range(5):
            r00 = a2_ref[pl.ds(qb + 2 * m * 16, 8, 2), :]
            r01 = a2_ref[pl.ds(qb + 2 * m * 16 + 1, 8, 2), :]
            r10 = a2_ref[pl.ds(qb + (2 * m + 1) * 16, 8, 2), :]
            r11 = a2_ref[pl.ds(qb + (2 * m + 1) * 16 + 1, 8, 2), :]
            p2_ref[co * _P2_ROWS + m * 8:co * _P2_ROWS + (m + 1) * 8, :] = \
                jnp.maximum(
                    jnp.maximum(jnp.maximum(r00, r01),
                                jnp.maximum(r10, r11)) + b2_ref[0, co],
                    0.0)

    p2 = p2_ref[...]
    h = jax.lax.dot_general(p2, wfc1_ref[...], (((0,), (0,)), ((), ())),
                            preferred_element_type=f32)
    h = jnp.maximum(h + bfc1_ref[...], 0.0)
    h = jnp.maximum(
        jnp.dot(h, wfc2_ref[...], preferred_element_type=f32) + bfc2_ref[...],
        0.0)
    out_ref[...] = (jnp.dot(h, wfc3_ref[...], preferred_element_type=f32)
                    + bfc3_ref[...])


def kernel(w1, b1, w2, b2, wfc1, bfc1, wfc2, bfc2, wfc3, bfc3, x_nchw):
    n = x_nchw.shape[0]
    nb = (n + _BLOCK - 1) // _BLOCK
    npad = nb * _BLOCK

    x = x_nchw.reshape(n, 28, 28).astype(jnp.float32)
    xp = jnp.pad(x, ((0, npad - n), (2, 2), (2, 2)))
    xT = jnp.pad(xp.reshape(npad, 1024).T, ((0, 160), (0, 0)))

    wf = jnp.transpose(wfc1.reshape(5, 5, 16, 128), (2, 0, 1, 3))
    wf = jnp.pad(wf, ((0, 0), (0, 0), (0, 3), (0, 0))).reshape(640, 128)

    eyes = jnp.stack([jnp.eye(_C2_ROWS, _P1_ROWS, k=di * 16 + dj,
                              dtype=jnp.float32)
                      for di in range(5) for dj in range(5)])
    w2d = jnp.einsum("tic,trs->cris", w2[:, :6, :], eyes)
    w2d = w2d.reshape(16 * _C2_ROWS, 6 * _P1_ROWS)
    w2d = w2d.astype(jnp.bfloat16)

    smem = pl.BlockSpec(memory_space=pltpu.SMEM)

    def _wspec(shp):
        return pl.BlockSpec(shp, lambda i, _s=shp: (0,) * len(_s))

    out = pl.pallas_call(
        _kern,
        grid=(nb,),
        out_shape=jax.ShapeDtypeStruct((npad, 128), jnp.float32),
        in_specs=[
            pl.BlockSpec((_IN_ROWS, _BLOCK), lambda i: (0, i)),
            smem,
            smem,
            _wspec((2560, 1536)),
            smem,
            _wspec((640, 128)),
            _wspec((1, 128)),
            _wspec((128, 128)),
            _wspec((1, 128)),
            _wspec((128, 128)),
            _wspec((1, 128)),
        ],
        out_specs=pl.BlockSpec((_BLOCK, 128), lambda i: (i, 0)),
        scratch_shapes=[
            pltpu.VMEM((2 * _C1_ROWS, _BLOCK), jnp.float32),
            pltpu.VMEM((6 * _P1_ROWS, _BLOCK), jnp.bfloat16),
            pltpu.VMEM((16 * _C2_ROWS, _BLOCK), jnp.float32),
            pltpu.VMEM((16 * _P2_ROWS, _BLOCK), jnp.float32),
        ],
        compiler_params=pltpu.CompilerParams(
            dimension_semantics=("parallel",)),
    )(xT, w1, b1, w2d, b2, wf, bfc1, wfc2, bfc2, wfc3, bfc3)
    return out[:n, :10]


# speedup vs baseline: 1.1440x; 1.0219x over previous
import jax
import jax.numpy as jnp
from jax.experimental import pallas as pl
from jax.experimental.pallas import tpu as pltpu

_BLOCK = 128
_IN_ROWS = 1184
_C1_ROWS = 1024
_P1_ROWS = 256
_C2_ROWS = 160
_P2_ROWS = 40
_M1 = 6 * 896
_K1 = 1056


def _kern(x_ref, w1_ref, b1_ref, w2d_ref, b2_ref,
          wfc1_ref, bfc1_ref, wfc2_ref, bfc2_ref, wfc3_ref, bfc3_ref,
          out_ref, a1_ref, p1_ref, a2_ref, p2_ref, xsh_ref):
    f32 = jnp.float32

    for p in range(1, 5):
        xsh_ref[(p - 1) * _IN_ROWS:(p - 1) * _IN_ROWS + 1032, :] = \
            x_ref[p:p + 1032, :]

    for cg in range(3):
        c0, c1 = 2 * cg, 2 * cg + 1
        for ch in range(7):
            base = ch * 128
            acc0 = jnp.full((128, _BLOCK), b1_ref[0, c0], f32)
            acc1 = jnp.full((128, _BLOCK), b1_ref[0, c1], f32)
            for di in range(5):
                for dj in range(5):
                    k = di * 5 + dj
                    if dj == 0:
                        xs = x_ref[base + di * 32:base + di * 32 + 128, :]
                    else:
                        o = (dj - 1) * _IN_ROWS + base + di * 32
                        xs = xsh_ref[o:o + 128, :]
                    acc0 = acc0 + xs * w1_ref[k, c0]
                    acc1 = acc1 + xs * w1_ref[k, c1]
            a1_ref[base:base + 128, :] = acc0
            a1_ref[_C1_ROWS + base:_C1_ROWS + base + 128, :] = acc1
        for ci in range(2):
            ab = ci * _C1_ROWS
            pb = (2 * cg + ci) * _P1_ROWS
            p1_ref[pb + 224:pb + 256, :] = jnp.zeros((32, _BLOCK),
                                                     jnp.bfloat16)
            for m in range(14):
                r00 = a1_ref[pl.ds(ab + 2 * m * 32, 16, 2), :]
                r01 = a1_ref[pl.ds(ab + 2 * m * 32 + 1, 16, 2), :]
                r10 = a1_ref[pl.ds(ab + (2 * m + 1) * 32, 16, 2), :]
                r11 = a1_ref[pl.ds(ab + (2 * m + 1) * 32 + 1, 16, 2), :]
                p1_ref[pb + m * 16:pb + (m + 1) * 16, :] = jnp.maximum(
                    jnp.maximum(jnp.maximum(r00, r01), jnp.maximum(r10, r11)),
                    0.0).astype(jnp.bfloat16)

    a2_ref[...] = jax.lax.dot_general(
        w2d_ref[...], p1_ref[...], (((1,), (0,)), ((), ())),
        preferred_element_type=f32)

    for co in range(16):
        qb = co * _C2_ROWS
        for m in range(5):
            r00 = a2_ref[pl.ds(qb + 2 * m * 16, 8, 2), :]
            r01 = a2_ref[pl.ds(qb + 2 * m * 16 + 1, 8, 2), :]
            r10 = a2_ref[pl.ds(qb + (2 * m + 1) * 16, 8, 2), :]
            r11 = a2_ref[pl.ds(qb + (2 * m + 1) * 16 + 1, 8, 2), :]
            p2_ref[co * _P2_ROWS + m * 8:co * _P2_ROWS + (m + 1) * 8, :] = \
                jnp.maximum(
                    jnp.maximum(jnp.maximum(r00, r01),
                                jnp.maximum(r10, r11)) + b2_ref[0, co],
                    0.0)

    p2 = p2_ref[...]
    h = jax.lax.dot_general(p2, wfc1_ref[...], (((0,), (0,)), ((), ())),
                            preferred_element_type=f32)
    h = jnp.maximum(h + bfc1_ref[...], 0.0)
    h = jnp.maximum(
        jnp.dot(h, wfc2_ref[...], preferred_element_type=f32) + bfc2_ref[...],
        0.0)
    out_ref[...] = (jnp.dot(h, wfc3_ref[...], preferred_element_type=f32)
                    + bfc3_ref[...])


def kernel(w1, b1, w2, b2, wfc1, bfc1, wfc2, bfc2, wfc3, bfc3, x_nchw):
    n = x_nchw.shape[0]
    nb = (n + _BLOCK - 1) // _BLOCK
    npad = nb * _BLOCK

    x = x_nchw.reshape(n, 28, 28).astype(jnp.float32)
    xp = jnp.pad(x, ((0, npad - n), (2, 2), (2, 2)))
    xT = jnp.pad(xp.reshape(npad, 1024).T, ((0, 160), (0, 0)))

    wf = jnp.transpose(wfc1.reshape(5, 5, 16, 128), (2, 0, 1, 3))
    wf = jnp.pad(wf, ((0, 0), (0, 0), (0, 3), (0, 0))).reshape(640, 128)

    eyes = jnp.stack([jnp.eye(_C2_ROWS, _P1_ROWS, k=di * 16 + dj,
                              dtype=jnp.float32)
                      for di in range(5) for dj in range(5)])
    w2d = jnp.einsum("tic,trs->cris", w2[:, :6, :], eyes)
    w2d = w2d.reshape(16 * _C2_ROWS, 6 * _P1_ROWS)
    w2d = w2d.astype(jnp.bfloat16)

    smem = pl.BlockSpec(memory_space=pltpu.SMEM)

    def _wspec(shp):
        return pl.BlockSpec(shp, lambda i, _s=shp: (0,) * len(_s))

    out = pl.pallas_call(
        _kern,
        grid=(nb,),
        out_shape=jax.ShapeDtypeStruct((npad, 128), jnp.float32),
        in_specs=[
            pl.BlockSpec((_IN_ROWS, _BLOCK), lambda i: (0, i)),
            smem,
            smem,
            _wspec((2560, 1536)),
            smem,
            _wspec((640, 128)),
            _wspec((1, 128)),
            _wspec((128, 128)),
            _wspec((1, 128)),
            _wspec((128, 128)),
            _wspec((1, 128)),
        ],
        out_specs=pl.BlockSpec((_BLOCK, 128), lambda i: (i, 0)),
        scratch_shapes=[
            pltpu.VMEM((2 * _C1_ROWS, _BLOCK), jnp.float32),
            pltpu.VMEM((6 * _P1_ROWS, _BLOCK), jnp.bfloat16),
            pltpu.VMEM((16 * _C2_ROWS, _BLOCK), jnp.float32),
            pltpu.VMEM((16 * _P2_ROWS, _BLOCK), jnp.float32),
            pltpu.VMEM((4 * _IN_ROWS, _BLOCK), jnp.float32),
        ],
        compiler_params=pltpu.CompilerParams(
            dimension_semantics=("parallel",)),
    )(xT, w1, b1, w2d, b2, wf, bfc1, wfc2, bfc2, wfc3, bfc3)
    return out[:n, :10]
